# baseline (speedup 1.0000x reference)
import functools

import jax
import jax.numpy as jnp
from jax import lax
from jax.experimental import pallas as pl
from jax.experimental.pallas import tpu as pltpu
from jax.experimental.pallas import tpu_sc as plsc

N = 10000
E = 320000
D = 128
NC = 2
NS = 16
NW = NC * NS
EPW = E // NW
CHUNK = 40
NCHUNK = EPW // CHUNK
ROWCH = 80
NRC = N // ROWCH


NBUF = 6


def _agg_body(x_hbm, src_hbm, dst_hbm, out_hbm,
              src_v, dst_v, b0, b1, b2, b3, b4, b5, table_sh,
              s0, s1, s2, s3, s4, s5):
    c = lax.axis_index("c")
    s = lax.axis_index("s")
    wid = s * NC + c
    bufs = (b0, b1, b2, b3, b4, b5)
    sems = (s0, s1, s2, s3, s4, s5)
    base = wid * EPW
    pltpu.sync_copy(src_hbm.at[pl.ds(base, EPW)], src_v)
    pltpu.sync_copy(dst_hbm.at[pl.ds(base, EPW)], dst_v)
    for b in range(NBUF):
        pltpu.async_copy(x_hbm.at[src_v.at[pl.ds(b * CHUNK, CHUNK)]],
                         bufs[b], sems[b])
    def zbody(j, carry):
        k = s + j * NS
        @pl.when(k < NRC)
        def _():
            pltpu.sync_copy(x_hbm.at[pl.ds(k * ROWCH, ROWCH)],
                            table_sh.at[pl.ds(k * ROWCH, ROWCH)])
        return carry
    lax.fori_loop(0, (NRC + NS - 1) // NS, zbody, 0)
    plsc.subcore_barrier()
    def ebody(i, carry):
        for b in range(NBUF):
            g = i * NBUF + b
            @pl.when(g < NCHUNK)
            def _():
                pltpu.make_async_copy(
                    x_hbm.at[src_v.at[pl.ds(g * CHUNK, CHUNK)]],
                    bufs[b], sems[b]).wait()
                pltpu.sync_copy(bufs[b],
                                table_sh.at[dst_v.at[pl.ds(g * CHUNK, CHUNK)]],
                                add=True)
                @pl.when(g + NBUF < NCHUNK)
                def _():
                    pltpu.async_copy(
                        x_hbm.at[src_v.at[pl.ds((g + NBUF) * CHUNK, CHUNK)]],
                        bufs[b], sems[b])
        return carry
    lax.fori_loop(0, (NCHUNK + NBUF - 1) // NBUF, ebody, 0)
    plsc.subcore_barrier()
    def wbody(j, carry):
        k = s + j * NS
        @pl.when(k < NRC)
        def _():
            pltpu.sync_copy(table_sh.at[pl.ds(k * ROWCH, ROWCH)],
                            out_hbm.at[c, pl.ds(k * ROWCH, ROWCH)])
        return carry
    lax.fori_loop(0, (NRC + NS - 1) // NS, wbody, 0)


_agg_call = pl.kernel(
    _agg_body,
    out_type=jax.ShapeDtypeStruct((NC, N, D), jnp.float32),
    mesh=plsc.VectorSubcoreMesh(core_axis_name="c", subcore_axis_name="s"),
    scratch_types=[
        pltpu.VMEM((EPW,), jnp.int32),
        pltpu.VMEM((EPW,), jnp.int32),
        pltpu.VMEM((CHUNK, D), jnp.float32),
        pltpu.VMEM((CHUNK, D), jnp.float32),
        pltpu.VMEM((CHUNK, D), jnp.float32),
        pltpu.VMEM((CHUNK, D), jnp.float32),
        pltpu.VMEM((CHUNK, D), jnp.float32),
        pltpu.VMEM((CHUNK, D), jnp.float32),
        pltpu.VMEM_SHARED((N, D), jnp.float32),
        pltpu.SemaphoreType.DMA,
        pltpu.SemaphoreType.DMA,
        pltpu.SemaphoreType.DMA,
        pltpu.SemaphoreType.DMA,
        pltpu.SemaphoreType.DMA,
        pltpu.SemaphoreType.DMA,
    ],
)


def _mlp_body(x_ref, a_ref, w1_ref, b1_ref, w2_ref, b2_ref, g_ref, be_ref, o_ref):
    h = (a_ref[0] + a_ref[1]) - x_ref[...]
    h = jnp.dot(h.astype(jnp.bfloat16), w1_ref[...].astype(jnp.bfloat16),
                preferred_element_type=jnp.float32) + b1_ref[...]
    h = jnp.maximum(h, 0.0)
    h = jnp.dot(h.astype(jnp.bfloat16), w2_ref[...].astype(jnp.bfloat16),
                preferred_element_type=jnp.float32) + b2_ref[...]
    h = jnp.maximum(h, 0.0)
    mean = jnp.mean(h, axis=0, keepdims=True)
    var = jnp.mean(jnp.square(h - mean), axis=0, keepdims=True)
    o_ref[...] = g_ref[...] * (h - mean) * lax.rsqrt(var + 1e-5) + be_ref[...]


_mlp_call = pl.pallas_call(
    _mlp_body,
    out_shape=jax.ShapeDtypeStruct((N, D), jnp.float32),
)


def kernel(x, edge_index, W1_0, b1_0, W2_0, b2_0, gamma_0, beta_0,
           W1_1, b1_1, W2_1, b2_1, gamma_1, beta_1,
           W1_2, b1_2, W2_2, b2_2, gamma_2, beta_2):
    src = edge_index[0]
    dst = edge_index[1]
    params = [
        (W1_0, b1_0, W2_0, b2_0, gamma_0, beta_0),
        (W1_1, b1_1, W2_1, b2_1, gamma_1, beta_1),
        (W1_2, b1_2, W2_2, b2_2, gamma_2, beta_2),
    ]
    for (W1, b1, W2, b2, g, b) in params:
        agg = _agg_call(x, src, dst)
        x = _mlp_call(x, agg, W1, b1[None, :], W2, b2[None, :],
                      g[None, :], b[None, :])
    return x


# speedup vs baseline: 1.0279x; 1.0279x over previous
import functools

import jax
import jax.numpy as jnp
from jax import lax
from jax.experimental import pallas as pl
from jax.experimental.pallas import tpu as pltpu
from jax.experimental.pallas import tpu_sc as plsc

N = 10000
E = 320000
D = 128
NC = 2
NS = 16
NW = NC * NS
EPW = E // NW
CHUNK = 40
NCHUNK = EPW // CHUNK
ROWCH = 80
NRC = N // ROWCH
NZC = N // CHUNK


NBUF = 6


def _agg_body(x_hbm, src_hbm, dst_hbm, out_hbm,
              src_v, dst_v, b0, b1, b2, b3, b4, b5, table_sh,
              s0, s1, s2, s3, s4, s5):
    c = lax.axis_index("c")
    s = lax.axis_index("s")
    wid = s * NC + c
    bufs = (b0, b1, b2, b3, b4, b5)
    sems = (s0, s1, s2, s3, s4, s5)
    base = wid * EPW
    pltpu.sync_copy(src_hbm.at[pl.ds(base, EPW)], src_v)
    pltpu.sync_copy(dst_hbm.at[pl.ds(base, EPW)], dst_v)
    def mbody(r, carry):
        for cc in range(D // 16):
            b0[r, pl.ds(cc * 16, 16)] = jnp.zeros((16,), jnp.float32)
        return carry
    lax.fori_loop(0, CHUNK, mbody, 0)
    def zbody(j, carry):
        k = s + j * NS
        @pl.when(k < NZC)
        def _():
            pltpu.sync_copy(b0, table_sh.at[pl.ds(k * CHUNK, CHUNK)])
        return carry
    lax.fori_loop(0, (NZC + NS - 1) // NS, zbody, 0)
    for b in range(NBUF):
        pltpu.async_copy(x_hbm.at[src_v.at[pl.ds(b * CHUNK, CHUNK)]],
                         bufs[b], sems[b])
    plsc.subcore_barrier()
    def ebody(i, carry):
        for b in range(NBUF):
            g = i * NBUF + b
            @pl.when(g < NCHUNK)
            def _():
                pltpu.make_async_copy(
                    x_hbm.at[src_v.at[pl.ds(g * CHUNK, CHUNK)]],
                    bufs[b], sems[b]).wait()
                pltpu.sync_copy(bufs[b],
                                table_sh.at[dst_v.at[pl.ds(g * CHUNK, CHUNK)]],
                                add=True)
                @pl.when(g + NBUF < NCHUNK)
                def _():
                    pltpu.async_copy(
                        x_hbm.at[src_v.at[pl.ds((g + NBUF) * CHUNK, CHUNK)]],
                        bufs[b], sems[b])
        return carry
    lax.fori_loop(0, (NCHUNK + NBUF - 1) // NBUF, ebody, 0)
    plsc.subcore_barrier()
    def wbody(j, carry):
        k = s + j * NS
        @pl.when(k < NRC)
        def _():
            pltpu.sync_copy(table_sh.at[pl.ds(k * ROWCH, ROWCH)],
                            out_hbm.at[c, pl.ds(k * ROWCH, ROWCH)])
        return carry
    lax.fori_loop(0, (NRC + NS - 1) // NS, wbody, 0)


_agg_call = pl.kernel(
    _agg_body,
    out_type=jax.ShapeDtypeStruct((NC, N, D), jnp.float32),
    mesh=plsc.VectorSubcoreMesh(core_axis_name="c", subcore_axis_name="s"),
    scratch_types=[
        pltpu.VMEM((EPW,), jnp.int32),
        pltpu.VMEM((EPW,), jnp.int32),
        pltpu.VMEM((CHUNK, D), jnp.float32),
        pltpu.VMEM((CHUNK, D), jnp.float32),
        pltpu.VMEM((CHUNK, D), jnp.float32),
        pltpu.VMEM((CHUNK, D), jnp.float32),
        pltpu.VMEM((CHUNK, D), jnp.float32),
        pltpu.VMEM((CHUNK, D), jnp.float32),
        pltpu.VMEM_SHARED((N, D), jnp.float32),
        pltpu.SemaphoreType.DMA,
        pltpu.SemaphoreType.DMA,
        pltpu.SemaphoreType.DMA,
        pltpu.SemaphoreType.DMA,
        pltpu.SemaphoreType.DMA,
        pltpu.SemaphoreType.DMA,
    ],
)


def _mlp_body(x_ref, a_ref, w1_ref, b1_ref, w2_ref, b2_ref, g_ref, be_ref, o_ref):
    h = x_ref[...] + a_ref[0] + a_ref[1]
    h = jnp.dot(h.astype(jnp.bfloat16), w1_ref[...].astype(jnp.bfloat16),
                preferred_element_type=jnp.float32) + b1_ref[...]
    h = jnp.maximum(h, 0.0)
    h = jnp.dot(h.astype(jnp.bfloat16), w2_ref[...].astype(jnp.bfloat16),
                preferred_element_type=jnp.float32) + b2_ref[...]
    h = jnp.maximum(h, 0.0)
    mean = jnp.mean(h, axis=0, keepdims=True)
    var = jnp.mean(jnp.square(h - mean), axis=0, keepdims=True)
    o_ref[...] = g_ref[...] * (h - mean) * lax.rsqrt(var + 1e-5) + be_ref[...]


_mlp_call = pl.pallas_call(
    _mlp_body,
    out_shape=jax.ShapeDtypeStruct((N, D), jnp.float32),
)


def kernel(x, edge_index, W1_0, b1_0, W2_0, b2_0, gamma_0, beta_0,
           W1_1, b1_1, W2_1, b2_1, gamma_1, beta_1,
           W1_2, b1_2, W2_2, b2_2, gamma_2, beta_2):
    src = edge_index[0]
    dst = edge_index[1]
    params = [
        (W1_0, b1_0, W2_0, b2_0, gamma_0, beta_0),
        (W1_1, b1_1, W2_1, b2_1, gamma_1, beta_1),
        (W1_2, b1_2, W2_2, b2_2, gamma_2, beta_2),
    ]
    for (W1, b1, W2, b2, g, b) in params:
        agg = _agg_call(x, src, dst)
        x = _mlp_call(x, agg, W1, b1[None, :], W2, b2[None, :],
                      g[None, :], b[None, :])
    return x
